# Initial kernel scaffold; baseline (speedup 1.0000x reference)
import jax
import jax.numpy as jnp
from jax.experimental import pallas as pl


def kernel(x, w1, b1, w2, b2, ws, bs):
    raise NotImplementedError("write your pallas kernel here")


# speedup vs baseline: 1.0068x; 1.0068x over previous
import functools

import jax
import jax.numpy as jnp
from jax.experimental import pallas as pl
from jax.experimental.pallas import tpu as pltpu

_VMEM_LIMIT = 32 * 1024 * 1024


def _scse_kernel(x_ref, wcat_ref, b1_ref, w2_ref, b2_ref, bs_ref, o_ref, *,
                 mid, inv_hw):
    xf = x_ref[0]

    res = jnp.dot(wcat_ref[...], xf,
                  preferred_element_type=jnp.float32)

    z = jnp.sum(res[0:mid, :], axis=1, keepdims=True) * inv_hw \
        + b1_ref[...]
    z = jnp.maximum(z, 0.0)
    s = jnp.dot(w2_ref[...], z,
                preferred_element_type=jnp.float32) + b2_ref[...]
    cse = jax.nn.sigmoid(s)

    sse = jax.nn.sigmoid(res[mid:mid + 1, :] + bs_ref[0])

    o_ref[0] = (xf * (cse + sse)).astype(o_ref.dtype)


def kernel(x, w1, b1, w2, b2, ws, bs):
    N, C, H, W = x.shape
    HW = H * W
    mid = w1.shape[0]

    x3 = x.reshape(N, C, HW)

    rows = mid + 1
    rows_pad = ((rows + 7) // 8) * 8
    wcat = jnp.concatenate(
        [w1.astype(jnp.float32), ws.reshape(1, C).astype(jnp.float32)], axis=0)
    wcat = jnp.pad(wcat, ((0, rows_pad - rows), (0, 0)))

    b1c = b1.reshape(mid, 1).astype(jnp.float32)
    b2c = b2.reshape(C, 1).astype(jnp.float32)
    bs_smem = bs.reshape(1).astype(jnp.float32)

    out = pl.pallas_call(
        functools.partial(_scse_kernel, mid=mid, inv_hw=1.0 / HW),
        out_shape=jax.ShapeDtypeStruct((N, C, HW), x.dtype),
        grid_spec=pltpu.PrefetchScalarGridSpec(
            num_scalar_prefetch=0,
            grid=(N,),
            in_specs=[
                pl.BlockSpec((1, C, HW), lambda n: (n, 0, 0)),
                pl.BlockSpec((rows_pad, C), lambda n: (0, 0)),
                pl.BlockSpec((mid, 1), lambda n: (0, 0)),
                pl.BlockSpec((C, mid), lambda n: (0, 0)),
                pl.BlockSpec((C, 1), lambda n: (0, 0)),
                pl.BlockSpec(memory_space=pltpu.MemorySpace.SMEM),
            ],
            out_specs=pl.BlockSpec((1, C, HW), lambda n: (n, 0, 0)),
        ),
        compiler_params=pltpu.CompilerParams(
            dimension_semantics=("parallel",),
            vmem_limit_bytes=_VMEM_LIMIT),
    )(x3, wcat, b1c, w2, b2c, bs_smem)

    return out.reshape(N, C, H, W)


# speedup vs baseline: 1.0574x; 1.0503x over previous
import functools

import jax
import jax.numpy as jnp
from jax.experimental import pallas as pl
from jax.experimental.pallas import tpu as pltpu

_VMEM_LIMIT = 32 * 1024 * 1024


def _scse_kernel(x_ref, wcat_ref, b1_ref, w2_ref, b2_ref, bs_ref, o_ref, *,
                 mid, inv_hw):
    xb = x_ref[0]

    res = jnp.dot(wcat_ref[...], xb,
                  preferred_element_type=jnp.float32)

    z = jnp.sum(res[0:mid, :], axis=1, keepdims=True) * inv_hw \
        + b1_ref[...]
    z = jnp.maximum(z, 0.0)
    s = jnp.dot(w2_ref[...], z,
                preferred_element_type=jnp.float32) + b2_ref[...]
    cse = jax.nn.sigmoid(s)

    sse = jax.nn.sigmoid(res[mid:mid + 1, :] + bs_ref[0])

    gate = cse.astype(jnp.bfloat16) + sse.astype(jnp.bfloat16)
    o_ref[0] = xb * gate


def kernel(x, w1, b1, w2, b2, ws, bs):
    N, C, H, W = x.shape
    HW = H * W
    mid = w1.shape[0]

    xb = x.reshape(N, C, HW).astype(jnp.bfloat16)

    rows = mid + 1
    rows_pad = ((rows + 7) // 8) * 8
    wcat = jnp.concatenate(
        [w1.astype(jnp.float32), ws.reshape(1, C).astype(jnp.float32)], axis=0)
    wcat = jnp.pad(wcat, ((0, rows_pad - rows), (0, 0))).astype(jnp.bfloat16)

    b1c = b1.reshape(mid, 1).astype(jnp.float32)
    b2c = b2.reshape(C, 1).astype(jnp.float32)
    bs_smem = bs.reshape(1).astype(jnp.float32)

    out = pl.pallas_call(
        functools.partial(_scse_kernel, mid=mid, inv_hw=1.0 / HW),
        out_shape=jax.ShapeDtypeStruct((N, C, HW), jnp.bfloat16),
        grid_spec=pltpu.PrefetchScalarGridSpec(
            num_scalar_prefetch=0,
            grid=(N,),
            in_specs=[
                pl.BlockSpec((1, C, HW), lambda n: (n, 0, 0)),
                pl.BlockSpec((rows_pad, C), lambda n: (0, 0)),
                pl.BlockSpec((mid, 1), lambda n: (0, 0)),
                pl.BlockSpec((C, mid), lambda n: (0, 0)),
                pl.BlockSpec((C, 1), lambda n: (0, 0)),
                pl.BlockSpec(memory_space=pltpu.MemorySpace.SMEM),
            ],
            out_specs=pl.BlockSpec((1, C, HW), lambda n: (n, 0, 0)),
        ),
        compiler_params=pltpu.CompilerParams(
            dimension_semantics=("parallel",),
            vmem_limit_bytes=_VMEM_LIMIT),
    )(xb, wcat, b1c, w2, b2c, bs_smem)

    return out.astype(jnp.float32).reshape(N, C, H, W)


# speedup vs baseline: 3.2298x; 3.0546x over previous
import functools

import jax
import jax.numpy as jnp
from jax.experimental import pallas as pl
from jax.experimental.pallas import tpu as pltpu

_VMEM_LIMIT = 32 * 1024 * 1024


def _scse_kernel(x_ref, w1t_ref, b1_ref, w2t_ref, b2_ref, ws_ref, bs_ref,
                 o_ref, *, inv_hw):
    xf = x_ref[0]

    pooled = jnp.sum(xf, axis=0, keepdims=True) * inv_hw
    z = jnp.dot(pooled, w1t_ref[...],
                preferred_element_type=jnp.float32) + b1_ref[...]
    z = jnp.maximum(z, 0.0)
    s = jnp.dot(z, w2t_ref[...],
                preferred_element_type=jnp.float32) + b2_ref[...]
    cse = jax.nn.sigmoid(s)

    sp = jnp.dot(xf, ws_ref[...],
                 preferred_element_type=jnp.float32) + bs_ref[0]
    sse = jax.nn.sigmoid(sp)

    o_ref[0] = xf * (cse + sse)


def kernel(x, w1, b1, w2, b2, ws, bs):
    N, C, H, W = x.shape
    HW = H * W
    mid = w1.shape[0]

    xt = jnp.transpose(x, (0, 2, 3, 1)).reshape(N, HW, C)

    w1t = w1.astype(jnp.float32).T
    w2t = w2.astype(jnp.float32).T
    b1r = b1.reshape(1, mid).astype(jnp.float32)
    b2r = b2.reshape(1, C).astype(jnp.float32)
    ws_col = ws.reshape(1, C).T.astype(jnp.float32)
    bs_smem = bs.reshape(1).astype(jnp.float32)

    out = pl.pallas_call(
        functools.partial(_scse_kernel, inv_hw=1.0 / HW),
        out_shape=jax.ShapeDtypeStruct((N, HW, C), jnp.float32),
        grid_spec=pltpu.PrefetchScalarGridSpec(
            num_scalar_prefetch=0,
            grid=(N,),
            in_specs=[
                pl.BlockSpec((1, HW, C), lambda n: (n, 0, 0)),
                pl.BlockSpec((C, mid), lambda n: (0, 0)),
                pl.BlockSpec((1, mid), lambda n: (0, 0)),
                pl.BlockSpec((mid, C), lambda n: (0, 0)),
                pl.BlockSpec((1, C), lambda n: (0, 0)),
                pl.BlockSpec((C, 1), lambda n: (0, 0)),
                pl.BlockSpec(memory_space=pltpu.MemorySpace.SMEM),
            ],
            out_specs=pl.BlockSpec((1, HW, C), lambda n: (n, 0, 0)),
        ),
        compiler_params=pltpu.CompilerParams(
            dimension_semantics=("parallel",),
            vmem_limit_bytes=_VMEM_LIMIT),
    )(xt, w1t, b1r, w2t, b2r, ws_col, bs_smem)

    return jnp.transpose(out.reshape(N, H, W, C), (0, 3, 1, 2))


# speedup vs baseline: 3.3395x; 1.0340x over previous
import functools

import jax
import jax.numpy as jnp
from jax.experimental import pallas as pl
from jax.experimental.pallas import tpu as pltpu

_VMEM_LIMIT = 48 * 1024 * 1024


def _scse_kernel(x_ref, w1t_ref, b1_ref, w2t_ref, b2_ref, ws_ref, bs_ref,
                 o_ref, *, hw, imgs, inv_hw):
    xf = x_ref[...]
    x2 = xf.reshape(imgs * hw, xf.shape[2])

    sp = jnp.dot(x2, ws_ref[...],
                 preferred_element_type=jnp.float32) + bs_ref[0]
    sse = jax.nn.sigmoid(sp).reshape(imgs, hw, 1)

    cses = []
    for i in range(imgs):
        pooled = jnp.sum(x2[i * hw:(i + 1) * hw], axis=0,
                         keepdims=True) * inv_hw
        z = jnp.dot(pooled, w1t_ref[...],
                    preferred_element_type=jnp.float32) + b1_ref[...]
        z = jnp.maximum(z, 0.0)
        s = jnp.dot(z, w2t_ref[...],
                    preferred_element_type=jnp.float32) + b2_ref[...]
        cses.append(jax.nn.sigmoid(s))
    cse = jnp.concatenate(cses, axis=0)[:, None, :]

    o_ref[...] = xf * (cse + sse)


def kernel(x, w1, b1, w2, b2, ws, bs):
    N, C, H, W = x.shape
    HW = H * W
    mid = w1.shape[0]
    B = 2

    xt = jnp.transpose(x, (0, 2, 3, 1)).reshape(N, HW, C)

    w1t = w1.astype(jnp.float32).T
    w2t = w2.astype(jnp.float32).T
    b1r = b1.reshape(1, mid).astype(jnp.float32)
    b2r = b2.reshape(1, C).astype(jnp.float32)
    ws_col = ws.reshape(1, C).T.astype(jnp.float32)
    bs_smem = bs.reshape(1).astype(jnp.float32)

    out = pl.pallas_call(
        functools.partial(_scse_kernel, hw=HW, imgs=B, inv_hw=1.0 / HW),
        out_shape=jax.ShapeDtypeStruct((N, HW, C), jnp.float32),
        grid_spec=pltpu.PrefetchScalarGridSpec(
            num_scalar_prefetch=0,
            grid=(N // B,),
            in_specs=[
                pl.BlockSpec((B, HW, C), lambda n: (n, 0, 0)),
                pl.BlockSpec((C, mid), lambda n: (0, 0)),
                pl.BlockSpec((1, mid), lambda n: (0, 0)),
                pl.BlockSpec((mid, C), lambda n: (0, 0)),
                pl.BlockSpec((1, C), lambda n: (0, 0)),
                pl.BlockSpec((C, 1), lambda n: (0, 0)),
                pl.BlockSpec(memory_space=pltpu.MemorySpace.SMEM),
            ],
            out_specs=pl.BlockSpec((B, HW, C), lambda n: (n, 0, 0)),
        ),
        compiler_params=pltpu.CompilerParams(
            dimension_semantics=("parallel",),
            vmem_limit_bytes=_VMEM_LIMIT),
    )(xt, w1t, b1r, w2t, b2r, ws_col, bs_smem)

    return jnp.transpose(out.reshape(N, H, W, C), (0, 3, 1, 2))
